# baseline (speedup 1.0000x reference)
import functools

import jax
import jax.numpy as jnp
from jax import lax
from jax.experimental import pallas as pl
from jax.experimental.pallas import tpu as pltpu
from jax.experimental.pallas import tpu_sc as plsc

_D = 64
_TOT = 4096 * 200
_IW = 128
_CHUNK = 640
_SUB = _CHUNK // _IW


@functools.cache
def _build():
    info = plsc.get_sparse_core_info()
    nw = info.num_cores * info.num_subcores
    b_per_w = _TOT // nw
    n_chunks = b_per_w // _CHUNK
    idx_rows = b_per_w // _IW

    mesh = plsc.VectorSubcoreMesh(core_axis_name="c", subcore_axis_name="s")

    @functools.partial(
        pl.kernel,
        mesh=mesh,
        out_type=jax.ShapeDtypeStruct((_TOT, _D), jnp.float32),
        scratch_types=[
            pltpu.VMEM((idx_rows, _IW), jnp.int32),
            pltpu.VMEM((_CHUNK, _D), jnp.float32),
            pltpu.SemaphoreType.DMA,
        ],
        compiler_params=pltpu.CompilerParams(use_tc_tiling_on_sc=False),
    )
    def gather_kernel(table_hbm, idx_hbm, out_hbm, idx_v, rows_v, gsem):
        wid = lax.axis_index("s") * info.num_cores + lax.axis_index("c")
        base = wid * b_per_w
        pltpu.sync_copy(idx_hbm.at[pl.ds(wid * idx_rows, idx_rows)], idx_v)

        def body(i, _):
            for j in range(_SUB):
                pltpu.async_copy(
                    table_hbm.at[idx_v.at[i * _SUB + j]],
                    rows_v.at[pl.ds(j * _IW, _IW)],
                    gsem,
                )
            for j in range(_SUB):
                pltpu.make_async_copy(
                    table_hbm.at[idx_v.at[i * _SUB + j]],
                    rows_v.at[pl.ds(j * _IW, _IW)],
                    gsem,
                ).wait()
            pltpu.sync_copy(rows_v, out_hbm.at[pl.ds(base + i * _CHUNK, _CHUNK)])
            return _

        lax.fori_loop(0, n_chunks, body, None)

    return gather_kernel


def kernel(input_ids, table):
    flat = input_ids.reshape(-1, _IW).astype(jnp.int32)
    out = _build()(table, flat)
    return out.reshape(input_ids.shape + (_D,))
